# baseline (speedup 1.0000x reference)
import jax
import jax.numpy as jnp
from jax import lax
from jax.experimental import pallas as pl
from jax.experimental.pallas import tpu as pltpu
from jax.experimental.pallas import tpu_sc as plsc

_BATCH = 1024
_EMB = 64
_VOCAB = 100000

_NUM_WORKERS = 32
_ROWS_PER_WORKER = _BATCH // _NUM_WORKERS

_VT = 2048
_NV = pl.cdiv(_VOCAB, _VT)
_VTA = 8192
_NVA = pl.cdiv(_VOCAB, _VTA)


def _sc_gather_pairs(table2, idx_half):
    mesh = plsc.VectorSubcoreMesh(core_axis_name="c", subcore_axis_name="s")

    @pl.kernel(
        mesh=mesh,
        out_type=jax.ShapeDtypeStruct((_BATCH, 2 * _EMB), table2.dtype),
        scratch_types=[
            pltpu.VMEM((_ROWS_PER_WORKER,), jnp.int32),
            pltpu.VMEM((_ROWS_PER_WORKER, 2 * _EMB), table2.dtype),
            pltpu.SemaphoreType.DMA,
        ],
    )
    def gather_kernel(table_hbm, idx_hbm, out_hbm, idx_v, rows_v, sem):
        wid = lax.axis_index("s") * 2 + lax.axis_index("c")
        base = wid * _ROWS_PER_WORKER
        pltpu.sync_copy(idx_hbm.at[pl.ds(base, _ROWS_PER_WORKER)], idx_v)
        pltpu.async_copy(table_hbm.at[idx_v], rows_v, sem).wait()
        pltpu.sync_copy(rows_v, out_hbm.at[pl.ds(base, _ROWS_PER_WORKER)])

    return gather_kernel(table2, idx_half)


def _select_augment(e2, par):
    e = jnp.where(par == 1, e2[:, _EMB:], e2[:, :_EMB])
    ones = jnp.ones((_BATCH, 1), jnp.float32)
    return jnp.concatenate([e, ones], axis=1).astype(jnp.bfloat16)


def _online_update(x, m_scr, s_scr):
    m_old = m_scr[...]
    m_new = jnp.maximum(m_old, jnp.max(x, axis=0, keepdims=True).astype(jnp.float32))
    e16 = jnp.exp(x - m_new.astype(jnp.bfloat16))
    blk = jnp.sum(e16.astype(jnp.float32), axis=0, keepdims=True)
    s_scr[...] = s_scr[...] * jnp.exp(m_old - m_new) + blk
    m_scr[...] = m_new


def _lse_body(e2_ref, p_ref, w_ref, o_ref, e_scr, m_scr, s_scr):
    j = pl.program_id(0)

    @pl.when(j == 0)
    def _init():
        e_scr[...] = _select_augment(e2_ref[...], p_ref[...])
        m_scr[...] = jnp.full((1, _BATCH), -1e30, jnp.float32)
        s_scr[...] = jnp.zeros((1, _BATCH), jnp.float32)

    x = lax.dot_general(
        w_ref[...], e_scr[...], (((0,), (1,)), ((), ())),
        preferred_element_type=jnp.float32,
    ).astype(jnp.bfloat16)

    @pl.when(j < _NVA - 1)
    def _full():
        _online_update(x, m_scr, s_scr)

    @pl.when(j == _NVA - 1)
    def _last():
        row = jax.lax.broadcasted_iota(jnp.int32, (_VTA, 1), 0) + j * _VTA
        _online_update(jnp.where(row < _VOCAB, x, jnp.bfloat16(-1e30)),
                       m_scr, s_scr)
        o_ref[...] = m_scr[...] + jnp.log(s_scr[...])


def _pass_a(e2, parity, w_aug):
    return pl.pallas_call(
        _lse_body,
        grid=(_NVA,),
        in_specs=[
            pl.BlockSpec((_BATCH, 2 * _EMB), lambda j: (0, 0)),
            pl.BlockSpec((_BATCH, 1), lambda j: (0, 0)),
            pl.BlockSpec((_EMB + 1, _VTA), lambda j: (0, j)),
        ],
        out_specs=pl.BlockSpec((1, _BATCH), lambda j: (0, 0)),
        out_shape=jax.ShapeDtypeStruct((1, _BATCH), jnp.float32),
        scratch_shapes=[
            pltpu.VMEM((_BATCH, _EMB + 1), jnp.bfloat16),
            pltpu.VMEM((1, _BATCH), jnp.float32),
            pltpu.VMEM((1, _BATCH), jnp.float32),
        ],
    )(e2, parity, w_aug)


def _write_body(e2_ref, p_ref, w_ref, l_ref, o_ref, e_scr):
    j = pl.program_id(0)

    @pl.when(j == 0)
    def _init():
        e_scr[...] = _select_augment(e2_ref[...], p_ref[...])

    x = lax.dot_general(
        w_ref[...], e_scr[...], (((0,), (1,)), ((), ())),
        preferred_element_type=jnp.float32,
    )
    o_ref[...] = x - l_ref[...]


def _pass_b(e2, parity, w_aug, lse):
    return pl.pallas_call(
        _write_body,
        grid=(_NV,),
        in_specs=[
            pl.BlockSpec((_BATCH, 2 * _EMB), lambda j: (0, 0)),
            pl.BlockSpec((_BATCH, 1), lambda j: (0, 0)),
            pl.BlockSpec((_EMB + 1, _VT), lambda j: (0, j)),
            pl.BlockSpec((1, _BATCH), lambda j: (0, 0)),
        ],
        out_specs=pl.BlockSpec((_VT, _BATCH), lambda j: (j, 0)),
        out_shape=jax.ShapeDtypeStruct((_VOCAB, _BATCH), jnp.float32),
        scratch_shapes=[
            pltpu.VMEM((_BATCH, _EMB + 1), jnp.bfloat16),
        ],
    )(e2, parity, w_aug, lse)


def kernel(input_word, emb_table, W, b):
    idx = input_word.astype(jnp.int32)
    table2 = emb_table.reshape(_VOCAB // 2, 2 * _EMB)
    e2 = _sc_gather_pairs(table2, idx >> 1)
    parity = (idx & 1).reshape(_BATCH, 1)
    w_aug = jnp.concatenate([W.T, b.reshape(1, _VOCAB)], axis=0)
    w_aug = w_aug.astype(jnp.bfloat16)
    lse = _pass_a(e2, parity, w_aug)
    out_t = _pass_b(e2, parity, w_aug, lse)
    return out_t.T


# speedup vs baseline: 1.0239x; 1.0239x over previous
import jax
import jax.numpy as jnp
from jax import lax
from jax.experimental import pallas as pl
from jax.experimental.pallas import tpu as pltpu
from jax.experimental.pallas import tpu_sc as plsc

_BATCH = 1024
_EMB = 64
_VOCAB = 100000

_NUM_WORKERS = 32
_ROWS_PER_WORKER = _BATCH // _NUM_WORKERS

_VT = 4096
_NV = pl.cdiv(_VOCAB, _VT)
_VTA = 4096
_NVA = pl.cdiv(_VOCAB, _VTA)


def _sc_gather_pairs(table2, idx_half):
    mesh = plsc.VectorSubcoreMesh(core_axis_name="c", subcore_axis_name="s")

    @pl.kernel(
        mesh=mesh,
        out_type=jax.ShapeDtypeStruct((_BATCH, 2 * _EMB), table2.dtype),
        scratch_types=[
            pltpu.VMEM((_ROWS_PER_WORKER,), jnp.int32),
            pltpu.VMEM((_ROWS_PER_WORKER, 2 * _EMB), table2.dtype),
            pltpu.SemaphoreType.DMA,
        ],
    )
    def gather_kernel(table_hbm, idx_hbm, out_hbm, idx_v, rows_v, sem):
        wid = lax.axis_index("s") * 2 + lax.axis_index("c")
        base = wid * _ROWS_PER_WORKER
        pltpu.sync_copy(idx_hbm.at[pl.ds(base, _ROWS_PER_WORKER)], idx_v)
        pltpu.async_copy(table_hbm.at[idx_v], rows_v, sem).wait()
        pltpu.sync_copy(rows_v, out_hbm.at[pl.ds(base, _ROWS_PER_WORKER)])

    return gather_kernel(table2, idx_half)


def _select_augment(e2, par):
    e = jnp.where(par == 1, e2[:, _EMB:], e2[:, :_EMB])
    ones = jnp.ones((_BATCH, 1), jnp.float32)
    return jnp.concatenate([e, ones], axis=1).astype(jnp.bfloat16)


def _online_update(x, m_scr, s_scr):
    m_old = m_scr[...]
    m_new = jnp.maximum(m_old, jnp.max(x, axis=0, keepdims=True))
    s_scr[...] = s_scr[...] * jnp.exp(m_old - m_new) + jnp.sum(
        jnp.exp(x - m_new), axis=0, keepdims=True)
    m_scr[...] = m_new


def _lse_body(e2_ref, p_ref, w_ref, o_ref, e_scr, m_scr, s_scr):
    j = pl.program_id(0)

    @pl.when(j == 0)
    def _init():
        e_scr[...] = _select_augment(e2_ref[...], p_ref[...])
        m_scr[...] = jnp.full((1, _BATCH), -1e30, jnp.float32)
        s_scr[...] = jnp.zeros((1, _BATCH), jnp.float32)

    x = lax.dot_general(
        w_ref[...], e_scr[...], (((0,), (1,)), ((), ())),
        preferred_element_type=jnp.float32,
    )

    @pl.when(j < _NVA - 1)
    def _full():
        _online_update(x, m_scr, s_scr)

    @pl.when(j == _NVA - 1)
    def _last():
        row = jax.lax.broadcasted_iota(jnp.int32, (_VTA, 1), 0) + j * _VTA
        _online_update(jnp.where(row < _VOCAB, x, -1e30), m_scr, s_scr)
        o_ref[...] = m_scr[...] + jnp.log(s_scr[...])


def _pass_a(e2, parity, w_aug):
    return pl.pallas_call(
        _lse_body,
        grid=(_NVA,),
        in_specs=[
            pl.BlockSpec((_BATCH, 2 * _EMB), lambda j: (0, 0)),
            pl.BlockSpec((_BATCH, 1), lambda j: (0, 0)),
            pl.BlockSpec((_EMB + 1, _VTA), lambda j: (0, j)),
        ],
        out_specs=pl.BlockSpec((1, _BATCH), lambda j: (0, 0)),
        out_shape=jax.ShapeDtypeStruct((1, _BATCH), jnp.float32),
        scratch_shapes=[
            pltpu.VMEM((_BATCH, _EMB + 1), jnp.bfloat16),
            pltpu.VMEM((1, _BATCH), jnp.float32),
            pltpu.VMEM((1, _BATCH), jnp.float32),
        ],
    )(e2, parity, w_aug)


def _write_body(e2_ref, p_ref, w_ref, l_ref, o_ref, e_scr):
    j = pl.program_id(0)

    @pl.when(j == 0)
    def _init():
        e_scr[...] = _select_augment(e2_ref[...], p_ref[...])

    x = lax.dot_general(
        w_ref[...], e_scr[...], (((0,), (1,)), ((), ())),
        preferred_element_type=jnp.float32,
    )
    o_ref[...] = x - l_ref[...]


def _pass_b(e2, parity, w_aug, lse):
    return pl.pallas_call(
        _write_body,
        grid=(_NV,),
        in_specs=[
            pl.BlockSpec((_BATCH, 2 * _EMB), lambda j: (0, 0)),
            pl.BlockSpec((_BATCH, 1), lambda j: (0, 0)),
            pl.BlockSpec((_EMB + 1, _VT), lambda j: (0, j)),
            pl.BlockSpec((1, _BATCH), lambda j: (0, 0)),
        ],
        out_specs=pl.BlockSpec((_VT, _BATCH), lambda j: (j, 0)),
        out_shape=jax.ShapeDtypeStruct((_VOCAB, _BATCH), jnp.float32),
        scratch_shapes=[
            pltpu.VMEM((_BATCH, _EMB + 1), jnp.bfloat16),
        ],
    )(e2, parity, w_aug, lse)


def kernel(input_word, emb_table, W, b):
    idx = input_word.astype(jnp.int32)
    table2 = emb_table.reshape(_VOCAB // 2, 2 * _EMB)
    e2 = _sc_gather_pairs(table2, idx >> 1)
    parity = (idx & 1).reshape(_BATCH, 1)
    w_aug = jnp.concatenate([W.T, b.reshape(1, _VOCAB)], axis=0)
    w_aug = w_aug.astype(jnp.bfloat16)
    lse = _pass_a(e2, parity, w_aug)
    out_t = _pass_b(e2, parity, w_aug, lse)
    return out_t.T


# speedup vs baseline: 1.0702x; 1.0453x over previous
import jax
import jax.numpy as jnp
from jax import lax
from jax.experimental import pallas as pl
from jax.experimental.pallas import tpu as pltpu
from jax.experimental.pallas import tpu_sc as plsc

_BATCH = 1024
_EMB = 64
_VOCAB = 100000

_NUM_WORKERS = 32
_ROWS_PER_WORKER = _BATCH // _NUM_WORKERS

_VT = 4096
_NV = pl.cdiv(_VOCAB, _VT)
_VTA = 4096
_NVA = pl.cdiv(_VOCAB, _VTA)


def _sc_gather_pairs(table2, idx_line):
    mesh = plsc.VectorSubcoreMesh(core_axis_name="c", subcore_axis_name="s")

    @pl.kernel(
        mesh=mesh,
        out_type=jax.ShapeDtypeStruct((_BATCH, 2 * _EMB), table2.dtype),
        scratch_types=[
            pltpu.VMEM((_ROWS_PER_WORKER,), jnp.int32),
            pltpu.VMEM((_ROWS_PER_WORKER, 2 * _EMB), table2.dtype),
            pltpu.SemaphoreType.DMA,
        ],
    )
    def gather_kernel(table_hbm, idx_hbm, out_hbm, idx_v, rows_v, sem):
        wid = lax.axis_index("s") * 2 + lax.axis_index("c")
        base = wid * _ROWS_PER_WORKER
        pltpu.sync_copy(idx_hbm.at[pl.ds(base, _ROWS_PER_WORKER)], idx_v)
        pltpu.async_copy(table_hbm.at[idx_v], rows_v, sem).wait()
        pltpu.sync_copy(rows_v, out_hbm.at[pl.ds(base, _ROWS_PER_WORKER)])

    return gather_kernel(table2, idx_line)


_RTH = 1024
_NRT = 49
_H = _RTH * _NRT


def _reformat_body(t1_ref, t2_ref, o_ref):
    o_ref[...] = jnp.concatenate([t1_ref[...].T, t2_ref[...].T], axis=1)


def _tc_reformat(table_t):
    return pl.pallas_call(
        _reformat_body,
        grid=(_NRT,),
        in_specs=[
            pl.BlockSpec((_EMB, _RTH), lambda j: (0, j)),
            pl.BlockSpec((_EMB, _RTH), lambda j: (0, j + _NRT)),
        ],
        out_specs=pl.BlockSpec((_RTH, 2 * _EMB), lambda j: (j, 0)),
        out_shape=jax.ShapeDtypeStruct((_H, 2 * _EMB), jnp.float32),
    )(table_t, table_t)


def _select_augment(e2, par):
    e = jnp.where(par == 1, e2[:, _EMB:], e2[:, :_EMB])
    ones = jnp.ones((_BATCH, 1), jnp.float32)
    return jnp.concatenate([e, ones], axis=1).astype(jnp.bfloat16)


def _online_update(x, m_scr, s_scr):
    m_old = m_scr[...]
    m_new = jnp.maximum(m_old, jnp.max(x, axis=0, keepdims=True))
    s_scr[...] = s_scr[...] * jnp.exp(m_old - m_new) + jnp.sum(
        jnp.exp(x - m_new), axis=0, keepdims=True)
    m_scr[...] = m_new


def _lse_body(e2_ref, p_ref, w_ref, o_ref, e_scr, m_scr, s_scr):
    j = pl.program_id(0)

    @pl.when(j == 0)
    def _init():
        e_scr[...] = _select_augment(e2_ref[...], p_ref[...])
        m_scr[...] = jnp.full((1, _BATCH), -1e30, jnp.float32)
        s_scr[...] = jnp.zeros((1, _BATCH), jnp.float32)

    x = lax.dot_general(
        w_ref[...], e_scr[...], (((0,), (1,)), ((), ())),
        preferred_element_type=jnp.float32,
    )

    @pl.when(j < _NVA - 1)
    def _full():
        _online_update(x, m_scr, s_scr)

    @pl.when(j == _NVA - 1)
    def _last():
        row = jax.lax.broadcasted_iota(jnp.int32, (_VTA, 1), 0) + j * _VTA
        _online_update(jnp.where(row < _VOCAB, x, -1e30), m_scr, s_scr)
        o_ref[...] = m_scr[...] + jnp.log(s_scr[...])


def _pass_a(e2, parity, w_aug):
    return pl.pallas_call(
        _lse_body,
        grid=(_NVA,),
        in_specs=[
            pl.BlockSpec((_BATCH, 2 * _EMB), lambda j: (0, 0)),
            pl.BlockSpec((_BATCH, 1), lambda j: (0, 0)),
            pl.BlockSpec((_EMB + 1, _VTA), lambda j: (0, j)),
        ],
        out_specs=pl.BlockSpec((1, _BATCH), lambda j: (0, 0)),
        out_shape=jax.ShapeDtypeStruct((1, _BATCH), jnp.float32),
        scratch_shapes=[
            pltpu.VMEM((_BATCH, _EMB + 1), jnp.bfloat16),
            pltpu.VMEM((1, _BATCH), jnp.float32),
            pltpu.VMEM((1, _BATCH), jnp.float32),
        ],
    )(e2, parity, w_aug)


def _write_body(e2_ref, p_ref, w_ref, l_ref, o_ref, e_scr):
    j = pl.program_id(0)

    @pl.when(j == 0)
    def _init():
        e_scr[...] = _select_augment(e2_ref[...], p_ref[...])

    x = lax.dot_general(
        w_ref[...], e_scr[...], (((0,), (1,)), ((), ())),
        preferred_element_type=jnp.float32,
    )
    o_ref[...] = x - l_ref[...]


def _pass_b(e2, parity, w_aug, lse):
    return pl.pallas_call(
        _write_body,
        grid=(_NV,),
        in_specs=[
            pl.BlockSpec((_BATCH, 2 * _EMB), lambda j: (0, 0)),
            pl.BlockSpec((_BATCH, 1), lambda j: (0, 0)),
            pl.BlockSpec((_EMB + 1, _VT), lambda j: (0, j)),
            pl.BlockSpec((1, _BATCH), lambda j: (0, 0)),
        ],
        out_specs=pl.BlockSpec((_VT, _BATCH), lambda j: (j, 0)),
        out_shape=jax.ShapeDtypeStruct((_VOCAB, _BATCH), jnp.float32),
        scratch_shapes=[
            pltpu.VMEM((_BATCH, _EMB + 1), jnp.bfloat16),
        ],
        compiler_params=pltpu.CompilerParams(
            dimension_semantics=("parallel",)),
    )(e2, parity, w_aug, lse)


def kernel(input_word, emb_table, W, b):
    idx = input_word.astype(jnp.int32)
    table2 = _tc_reformat(emb_table.T)
    hi = idx >= _H
    e2 = _sc_gather_pairs(table2, jnp.where(hi, idx - _H, idx))
    parity = hi.astype(jnp.int32).reshape(_BATCH, 1)
    w_aug = jnp.concatenate([W.T, b.reshape(1, _VOCAB)], axis=0)
    w_aug = w_aug.astype(jnp.bfloat16)
    lse = _pass_a(e2, parity, w_aug)
    out_t = _pass_b(e2, parity, w_aug, lse)
    return out_t.T
